# Initial kernel scaffold; baseline (speedup 1.0000x reference)
import jax
import jax.numpy as jnp
from jax.experimental import pallas as pl


def kernel(mol_x, protein_x, params, mol_edge_index, mol_batch, protein_edge_index):
    raise NotImplementedError("write your pallas kernel here")


# speedup vs baseline: 14.1323x; 14.1323x over previous
import functools

import jax
import jax.numpy as jnp
from jax import lax
from jax.experimental import pallas as pl
from jax.experimental.pallas import tpu as pltpu
from jax.experimental.pallas import tpu_sc as plsc

HID = 128
HEADS = 4
OUTC = 32
NGRAPHS = 256
L = 16
NC = 2
NS = 16
NW = NC * NS
EB = 64
BLK = 512

N_M = 50000
E_M = 800000
NPAD_M = 51200
NBUCK_M = 16
CH_M = NPAD_M // NBUCK_M
CAP_M = E_M + NW * EB

N_P = 10000
E_P = 160000
NPAD_P = 10240
NBUCK_P = 2
CH_P = NPAD_P // NBUCK_P
CAP_P = E_P + NW * EB

_SC_PARAMS = pltpu.CompilerParams(needs_layout_passes=False)
NEG = -3.4e38


def _vext(vec, j):
    m = lax.broadcasted_iota(jnp.int32, (L,), 0) == j
    return jnp.max(jnp.where(m, vec, 0))


def _make_sc_deg(npad, nbuck, ch, cap):
    mesh = plsc.VectorSubcoreMesh(core_axis_name="c", subcore_axis_name="s")
    per_sc = nbuck // NC
    rpw = ch // NS

    @functools.partial(
        pl.kernel,
        out_type=jax.ShapeDtypeStruct((npad,), jnp.float32),
        mesh=mesh,
        compiler_params=_SC_PARAMS,
        scratch_types=[
            pltpu.VMEM((EB,), jnp.int32),
            pltpu.VMEM((EB,), jnp.float32),
            pltpu.VMEM((L,), jnp.float32),
            pltpu.VMEM((L,), jnp.int32),
            pltpu.VMEM((ch // NS,), jnp.float32),
            pltpu.VMEM_SHARED((ch + 2 * L,), jnp.float32),
            pltpu.SemaphoreType.DMA,
        ],
    )
    def deg_kernel(edstl_hbm, meta_hbm, deg_hbm, didx, ones, zv, meta, fbuf,
                   deg_sh, sem):
        cid = lax.axis_index("c")
        sid = lax.axis_index("s")

        def init(i, _):
            ones[pl.ds(i * L, L)] = jnp.ones((L,), jnp.float32)
            return 0

        lax.fori_loop(0, EB // L, init, 0)
        zv[pl.ds(0, L)] = jnp.zeros((L,), jnp.float32)
        pltpu.sync_copy(meta_hbm, meta)
        mv = meta[:]
        for kk in range(per_sc):
            k = cid + kk * NC
            nb = _vext(mv, k)
            nz = ch // L

            def zero(i, _):
                jj = sid + i * NS

                @pl.when(jj < nz)
                def _():
                    pltpu.sync_copy(zv, deg_sh.at[pl.ds(jj * L, L)])

                return 0

            lax.fori_loop(0, (nz + NS - 1) // NS, zero, 0)
            plsc.subcore_barrier()
            base = k * cap + sid * nb * EB

            def body(b, _):
                pltpu.sync_copy(edstl_hbm.at[pl.ds(base + b * EB, EB)], didx)
                pltpu.sync_copy(ones, deg_sh.at[didx], add=True)
                return 0

            lax.fori_loop(0, nb, body, 0)
            plsc.subcore_barrier()
            pltpu.sync_copy(deg_sh.at[pl.ds(sid * rpw, rpw)], fbuf)
            pltpu.sync_copy(fbuf, deg_hbm.at[pl.ds(k * ch + sid * rpw, rpw)])
            plsc.subcore_barrier()

    return deg_kernel


def _make_sc_gcn(npad, nbuck, ch, cap):
    mesh = plsc.VectorSubcoreMesh(core_axis_name="c", subcore_axis_name="s")
    per_sc = nbuck // NC
    rpw = ch // NS

    @functools.partial(
        pl.kernel,
        out_type=jax.ShapeDtypeStruct((npad, HID), jnp.float32),
        mesh=mesh,
        compiler_params=_SC_PARAMS,
        scratch_types=[
            pltpu.VMEM((EB,), jnp.int32),
            pltpu.VMEM((EB,), jnp.int32),
            pltpu.VMEM((EB, HID), jnp.float32),
            pltpu.VMEM((L, HID), jnp.float32),
            pltpu.VMEM((L,), jnp.int32),
            pltpu.VMEM_SHARED((ch + 2 * L, HID), jnp.float32),
            pltpu.SemaphoreType.DMA,
        ],
    )
    def gcn_kernel(hs_hbm, esrc_hbm, edstl_hbm, meta_hbm, acc_hbm,
                   sidx, didx, rows, zbuf, meta, acc_sh, sem):
        cid = lax.axis_index("c")
        sid = lax.axis_index("s")

        def zinit(i, _):
            for j in range(HID // L):
                zbuf[i, pl.ds(j * L, L)] = jnp.zeros((L,), jnp.float32)
            return 0

        lax.fori_loop(0, L, zinit, 0)
        pltpu.sync_copy(meta_hbm, meta)
        mv = meta[:]
        for kk in range(per_sc):
            k = cid + kk * NC
            nb = _vext(mv, k)
            nz = ch // L

            def zero(i, _):
                jj = sid + i * NS

                @pl.when(jj < nz)
                def _():
                    pltpu.sync_copy(zbuf, acc_sh.at[pl.ds(jj * L, L)])

                return 0

            lax.fori_loop(0, (nz + NS - 1) // NS, zero, 0)
            plsc.subcore_barrier()
            base = k * cap + sid * nb * EB

            def body(b, _):
                off = base + b * EB
                pltpu.sync_copy(esrc_hbm.at[pl.ds(off, EB)], sidx)
                pltpu.sync_copy(edstl_hbm.at[pl.ds(off, EB)], didx)
                pltpu.async_copy(hs_hbm.at[sidx], rows, sem).wait()
                pltpu.sync_copy(rows, acc_sh.at[didx], add=True)
                return 0

            lax.fori_loop(0, nb, body, 0)
            plsc.subcore_barrier()
            pltpu.sync_copy(
                acc_sh.at[pl.ds(sid * rpw, rpw)],
                acc_hbm.at[pl.ds(k * ch + sid * rpw, rpw)],
            )
            plsc.subcore_barrier()

    return gcn_kernel


HX = 2 * HID


def _make_sc_gat(npad, nbuck, ch, cap):
    mesh = plsc.VectorSubcoreMesh(core_axis_name="c", subcore_axis_name="s")
    per_sc = nbuck // NC
    rpw = ch // NS

    @functools.partial(
        pl.kernel,
        out_type=[
            jax.ShapeDtypeStruct((npad, HID), jnp.float32),
            jax.ShapeDtypeStruct((npad, HID), jnp.float32),
        ],
        mesh=mesh,
        compiler_params=_SC_PARAMS,
        scratch_types=[
            pltpu.VMEM((EB,), jnp.int32),
            pltpu.VMEM((EB,), jnp.int32),
            pltpu.VMEM((EB,), jnp.int32),
            pltpu.VMEM((EB, HX), jnp.float32),
            pltpu.VMEM((EB, HID), jnp.float32),
            pltpu.VMEM((EB, HID), jnp.float32),
            pltpu.VMEM((EB, HID), jnp.float32),
            pltpu.VMEM((L, HID), jnp.float32),
            pltpu.VMEM((L,), jnp.int32),
            pltpu.VMEM_SHARED((ch + 2 * L, HID), jnp.float32),
            pltpu.VMEM_SHARED((ch + 2 * L, HID), jnp.float32),
            pltpu.SemaphoreType.DMA,
        ],
    )
    def gat_kernel(hext_hbm, esed_hbm, esrc_hbm, edstl_hbm, meta_hbm,
                   acc_hbm, aex_hbm,
                   sidx, didx, dgix, rows, edr, rowsa, rowsb, zbuf, meta,
                   acc_sh, aex_sh, sem):
        cid = lax.axis_index("c")
        sid = lax.axis_index("s")

        def zinit(i, _):
            for j in range(HID // L):
                zbuf[i, pl.ds(j * L, L)] = jnp.zeros((L,), jnp.float32)
            return 0

        lax.fori_loop(0, L, zinit, 0)

        def zb_init(e, _):
            for j in range(HID // L):
                rowsb[e, pl.ds(j * L, L)] = jnp.zeros((L,), jnp.float32)
            return 0

        lax.fori_loop(0, EB, zb_init, 0)
        pltpu.sync_copy(meta_hbm, meta)
        mv = meta[:]
        hiota = lax.broadcasted_iota(jnp.int32, (L,), 0)
        shvec = jnp.minimum(HEADS + hiota, HID - 1)
        for kk in range(per_sc):
            k = cid + kk * NC
            nb = _vext(mv, k)
            nz = ch // L

            def zero(i, _):
                jj = sid + i * NS

                @pl.when(jj < nz)
                def _():
                    pltpu.sync_copy(zbuf, acc_sh.at[pl.ds(jj * L, L)])
                    pltpu.sync_copy(zbuf, aex_sh.at[pl.ds(jj * L, L)])

                return 0

            lax.fori_loop(0, (nz + NS - 1) // NS, zero, 0)
            plsc.subcore_barrier()
            base = k * cap + sid * nb * EB

            def body(b, _):
                off = base + b * EB
                pltpu.sync_copy(esrc_hbm.at[pl.ds(off, EB)], sidx)
                pltpu.sync_copy(edstl_hbm.at[pl.ds(off, EB)], didx)

                def mkglobal(j, _):
                    v = didx[pl.ds(j * L, L)]
                    dgix[pl.ds(j * L, L)] = jnp.minimum(v + k * ch, npad - 1)
                    return 0

                lax.fori_loop(0, EB // L, mkglobal, 0)
                pltpu.async_copy(hext_hbm.at[sidx], rows, sem).wait()
                pltpu.async_copy(esed_hbm.at[dgix], edr, sem).wait()

                def edge(e, _):
                    edv = plsc.load_gather(
                        edr, [jnp.full((L,), e, jnp.int32), shvec])
                    ev = rows[e, pl.ds(HID, L)] + edv
                    lv = jnp.maximum(ev, 0.2 * ev)
                    xv = jnp.exp(lv)
                    rowsb[e, pl.ds(0, L)] = xv
                    for hd in range(HEADS):
                        s = jnp.max(jnp.where(hiota == hd, xv, NEG))
                        c0 = hd * OUTC
                        rowsa[e, pl.ds(c0, L)] = rows[e, pl.ds(c0, L)] * s
                        rowsa[e, pl.ds(c0 + L, L)] = rows[e, pl.ds(c0 + L, L)] * s
                    return 0

                lax.fori_loop(0, EB, edge, 0)
                pltpu.sync_copy(rowsa, acc_sh.at[didx], add=True)
                pltpu.sync_copy(rowsb, aex_sh.at[didx], add=True)
                return 0

            lax.fori_loop(0, nb, body, 0)
            plsc.subcore_barrier()
            pltpu.sync_copy(
                acc_sh.at[pl.ds(sid * rpw, rpw)],
                acc_hbm.at[pl.ds(k * ch + sid * rpw, rpw)],
            )
            pltpu.sync_copy(
                aex_sh.at[pl.ds(sid * rpw, rpw)],
                aex_hbm.at[pl.ds(k * ch + sid * rpw, rpw)],
            )
            plsc.subcore_barrier()

    return gat_kernel


def _tc_dinv(deg2d, n):
    r, c = deg2d.shape

    def body(deg_ref, out_ref):
        gid = (lax.broadcasted_iota(jnp.int32, (r, c), 0) * c
               + lax.broadcasted_iota(jnp.int32, (r, c), 1))
        d = deg_ref[...]
        out_ref[...] = jnp.where(gid < n, lax.rsqrt(d + 1.0), 0.0)

    return pl.pallas_call(
        body, out_shape=jax.ShapeDtypeStruct((r, c), jnp.float32)
    )(deg2d)


def _tc_gcn_layer(npad, kin, first):
    nb = npad // BLK
    rspec = pl.BlockSpec((BLK, kin), lambda i: (i, 0))
    hspec = pl.BlockSpec((BLK, HID), lambda i: (i, 0))
    dspec = pl.BlockSpec((BLK, 1), lambda i: (i, 0))
    wspec = pl.BlockSpec((kin, HID), lambda i: (0, 0))
    bspec = pl.BlockSpec((1, HID), lambda i: (0, 0))

    if first:
        def body(x_ref, dinv_ref, w_ref, out_ref):
            h = jnp.dot(x_ref[...], w_ref[...], preferred_element_type=jnp.float32)
            out_ref[...] = h * dinv_ref[...]

        in_specs = [rspec, dspec, wspec]
    else:
        def body(acc_ref, hs_ref, dinv_ref, b_ref, w_ref, out_ref):
            t = jax.nn.relu((acc_ref[...] + hs_ref[...]) * dinv_ref[...] + b_ref[...])
            h = jnp.dot(t, w_ref[...], preferred_element_type=jnp.float32)
            out_ref[...] = h * dinv_ref[...]

        in_specs = [hspec, hspec, dspec, bspec, wspec]

    return pl.pallas_call(
        body,
        grid=(nb,),
        in_specs=in_specs,
        out_specs=hspec,
        out_shape=jax.ShapeDtypeStruct((npad, HID), jnp.float32),
    )


def _tc_to_gat(npad, from_gat):
    nb = npad // BLK
    hspec = pl.BlockSpec((BLK, HID), lambda i: (i, 0))
    xspec = pl.BlockSpec((BLK, HX), lambda i: (i, 0))
    dspec = pl.BlockSpec((BLK, 1), lambda i: (i, 0))
    wspec = pl.BlockSpec((HID, HID), lambda i: (0, 0))
    bspec = pl.BlockSpec((1, HID), lambda i: (0, 0))

    if from_gat:
        def body(acc_ref, aex_ref, hext_ref, b_ref, w_ref, a_ref,
                 out_hext, out_esed):
            t = _gat_combine(acc_ref[...], aex_ref[...], hext_ref[...], b_ref[...])
            _emit_gat_head(t, w_ref, a_ref, out_hext, out_esed)

        in_specs = [hspec, hspec, xspec, bspec, wspec, wspec]
    else:
        def body(acc_ref, hs_ref, dinv_ref, b_ref, w_ref, a_ref,
                 out_hext, out_esed):
            t = jax.nn.relu((acc_ref[...] + hs_ref[...]) * dinv_ref[...] + b_ref[...])
            _emit_gat_head(t, w_ref, a_ref, out_hext, out_esed)

        in_specs = [hspec, hspec, dspec, bspec, wspec, wspec]

    return pl.pallas_call(
        body,
        grid=(nb,),
        in_specs=in_specs,
        out_specs=[xspec, hspec],
        out_shape=[
            jax.ShapeDtypeStruct((npad, HX), jnp.float32),
            jax.ShapeDtypeStruct((npad, HID), jnp.float32),
        ],
    )


def _emit_gat_head(t, w_ref, a_ref, out_hext, out_esed):
    h = jnp.dot(t, w_ref[...], preferred_element_type=jnp.float32)
    esed = jnp.dot(h, a_ref[...], preferred_element_type=jnp.float32)
    out_hext[...] = jnp.concatenate([h, esed], axis=1)
    out_esed[...] = esed


def _gat_combine(acc, aex, hext, b):
    parts = []
    for hd in range(HEADS):
        c0 = hd * OUTC
        e_self = (hext[:, HID + hd:HID + hd + 1]
                  + hext[:, HID + HEADS + hd:HID + HEADS + hd + 1])
        exh = jnp.exp(jnp.maximum(e_self, 0.2 * e_self))
        numc = acc[:, c0:c0 + OUTC] + exh * hext[:, c0:c0 + OUTC]
        denc = aex[:, hd:hd + 1] + exh + 1e-16
        parts.append(numc / denc)
    return jax.nn.relu(jnp.concatenate(parts, axis=1) + b)


def _tc_gat_final(npad):
    nb = npad // BLK
    hspec = pl.BlockSpec((BLK, HID), lambda i: (i, 0))
    xspec = pl.BlockSpec((BLK, HX), lambda i: (i, 0))
    bspec = pl.BlockSpec((1, HID), lambda i: (0, 0))

    def body(acc_ref, aex_ref, hext_ref, b_ref, out_ref):
        out_ref[...] = _gat_combine(acc_ref[...], aex_ref[...],
                                    hext_ref[...], b_ref[...])

    return pl.pallas_call(
        body,
        grid=(nb,),
        in_specs=[hspec, hspec, xspec, bspec],
        out_specs=hspec,
        out_shape=jax.ShapeDtypeStruct((npad, HID), jnp.float32),
    )


def _tc_pool(h, starts):
    npad = h.shape[0]

    def body(starts_ref, h_ref, sum_ref, max_ref, cnt_ref):
        g = pl.program_id(0)
        s = starts_ref[g]
        e = starts_ref[g + 1]
        n = e - s
        iters = (n + 7) // 8

        def it(i, carry):
            s8, m8 = carry
            blk = h_ref[pl.ds(s + i * 8, 8), :]
            rowid = s + i * 8 + lax.broadcasted_iota(jnp.int32, (8, HID), 0)
            ok = rowid < e
            s8 = s8 + jnp.where(ok, blk, 0.0)
            m8 = jnp.maximum(m8, jnp.where(ok, blk, NEG))
            return s8, m8

        s8 = jnp.zeros((8, HID), jnp.float32)
        m8 = jnp.full((8, HID), NEG, jnp.float32)
        s8, m8 = lax.fori_loop(0, iters, it, (s8, m8))
        sum_ref[...] = jnp.sum(s8, axis=0, keepdims=True)[None]
        max_ref[...] = jnp.max(m8, axis=0, keepdims=True)[None]
        cnt_ref[...] = jnp.full((1, 1, HID), n, jnp.float32)

    ospec = pl.BlockSpec((1, 1, HID), lambda g: (g, 0, 0))
    oshape = jax.ShapeDtypeStruct((NGRAPHS, 1, HID), jnp.float32)
    outs = pl.pallas_call(
        body,
        grid=(NGRAPHS,),
        in_specs=[
            pl.BlockSpec(memory_space=pltpu.SMEM),
            pl.BlockSpec((npad, HID), lambda g: (0, 0)),
        ],
        out_specs=[ospec, ospec, ospec],
        out_shape=[oshape, oshape, oshape],
        compiler_params=pltpu.CompilerParams(vmem_limit_bytes=100 * 1024 * 1024),
    )(starts, h)
    return tuple(o.reshape(NGRAPHS, HID) for o in outs)


def _tc_psum(px, n):
    npad = px.shape[0]
    nb = npad // BLK

    def body(x_ref, out_ref):
        i = pl.program_id(0)

        @pl.when(i == 0)
        def _():
            out_ref[...] = jnp.zeros((1, HID), jnp.float32)

        rowid = i * BLK + lax.broadcasted_iota(jnp.int32, (BLK, HID), 0)
        blk = jnp.where(rowid < n, x_ref[...], 0.0)
        out_ref[...] += jnp.sum(blk, axis=0, keepdims=True)

    return pl.pallas_call(
        body,
        grid=(nb,),
        in_specs=[pl.BlockSpec((BLK, HID), lambda i: (i, 0))],
        out_specs=pl.BlockSpec((1, HID), lambda i: (0, 0)),
        out_shape=jax.ShapeDtypeStruct((1, HID), jnp.float32),
    )(px)


def _tc_head(msum, mmax, cnt, psum, w):
    def body(msum_ref, mmax_ref, cnt_ref, psum_ref,
             wv_ref, bv_ref, wo_ref, bo_ref,
             jt1w_ref, jt1b_ref, jt2w_ref, jt2b_ref,
             pl0w_ref, pl0b_ref, pl1w_ref, pl1b_ref, pl2w_ref, pl2b_ref,
             pr1w_ref, pr1b_ref, pr2w_ref, pr2b_ref,
             pr3w_ref, pr3b_ref, pr4w_ref, pr4b_ref, out_ref):
        f32 = jnp.float32
        cntv = cnt_ref[...]
        msumv = msum_ref[...]
        mmean = msumv / jnp.maximum(cntv, 1.0)
        mmaxv = jnp.where(cntv > 0, mmax_ref[...], 0.0)
        pg = jnp.broadcast_to(psum_ref[...] / float(N_P), (NGRAPHS, HID))

        def mm(x, wr, br):
            return jnp.dot(x, wr[...], preferred_element_type=f32) + br[...]

        att = mm(mm(pg, wv_ref, bv_ref), wo_ref, bo_ref)
        jf = jnp.concatenate([att, pg, mmean], axis=1)
        jf = jax.nn.relu(mm(jf, jt1w_ref, jt1b_ref))
        jf = jax.nn.relu(mm(jf, jt2w_ref, jt2b_ref))
        p0 = mm(mmean, pl0w_ref, pl0b_ref)
        p1 = mm(mmaxv, pl1w_ref, pl1b_ref)
        p2 = mm(msumv, pl2w_ref, pl2b_ref)
        ff = jnp.concatenate([jf, p0, p1, p2], axis=1)
        o = jax.nn.relu(mm(ff, pr1w_ref, pr1b_ref))
        o = jax.nn.relu(mm(o, pr2w_ref, pr2b_ref))
        o = jax.nn.relu(mm(o, pr3w_ref, pr3b_ref))
        o = jax.nn.sigmoid(mm(o, pr4w_ref, pr4b_ref))
        out_ref[...] = jnp.broadcast_to(o, (NGRAPHS, HID))

    args = [msum, mmax, cnt, psum,
            w['Wv'], w['bv'][None, :], w['Wo'], w['bo'][None, :],
            w['jt1_W'], w['jt1_b'][None, :], w['jt2_W'], w['jt2_b'][None, :],
            w['pl0_W'], w['pl0_b'][None, :], w['pl1_W'], w['pl1_b'][None, :],
            w['pl2_W'], w['pl2_b'][None, :],
            w['pr1_W'], w['pr1_b'][None, :], w['pr2_W'], w['pr2_b'][None, :],
            w['pr3_W'], w['pr3_b'][None, :], w['pr4_W'], w['pr4_b'][None, :]]
    out = pl.pallas_call(
        body, out_shape=jax.ShapeDtypeStruct((NGRAPHS, HID), jnp.float32)
    )(*args)
    return out[:, 0]


def _bucketize(src, dst, e, nbuck, ch, cap):
    b = dst // ch
    order = jnp.argsort(b, stable=True)
    srcs = src[order]
    dsts = dst[order]
    bs = b[order]
    cnt = jnp.zeros((nbuck,), jnp.int32).at[b].add(1)
    start = jnp.concatenate([jnp.zeros((1,), jnp.int32),
                             jnp.cumsum(cnt)[:-1].astype(jnp.int32)])
    rank = jnp.arange(e, dtype=jnp.int32) - start[bs]
    pos = bs * cap + rank
    nb_w = (cnt + NS * EB - 1) // (NS * EB)
    esrc = jnp.zeros((nbuck * cap,), jnp.int32).at[pos].set(srcs)
    edstl = jnp.full((nbuck * cap,), ch, jnp.int32).at[pos].set(dsts - bs * ch)
    meta = jnp.zeros((L,), jnp.int32).at[:nbuck].set(nb_w.astype(jnp.int32))
    return esrc, edstl, meta


def _attn_mat(a_s, a_d):
    A = jnp.zeros((HID, HID), jnp.float32)
    for hd in range(HEADS):
        A = A.at[hd * OUTC:(hd + 1) * OUTC, hd].set(a_s[hd])
        A = A.at[hd * OUTC:(hd + 1) * OUTC, HEADS + hd].set(a_d[hd])
    return A


def _pad_rows(x, npad):
    return jnp.pad(x, ((0, npad - x.shape[0]), (0, 0)))


def kernel(mol_x, protein_x, params, mol_edge_index, mol_batch, protein_edge_index):
    p = params
    i32 = jnp.int32

    ms = mol_edge_index[0].astype(i32)
    md = mol_edge_index[1].astype(i32)
    ps = protein_edge_index[0].astype(i32)
    pd = protein_edge_index[1].astype(i32)

    esrc_m, edstl_m, meta_m = _bucketize(ms, md, E_M, NBUCK_M, CH_M, CAP_M)
    esrc_p, edstl_p, meta_p = _bucketize(ps, pd, E_P, NBUCK_P, CH_P, CAP_P)
    starts = jnp.searchsorted(mol_batch.astype(i32),
                              jnp.arange(NGRAPHS + 1, dtype=i32)).astype(i32)

    xm = _pad_rows(mol_x.astype(jnp.float32), NPAD_M)
    xp = jnp.pad(protein_x.astype(jnp.float32),
                 ((0, NPAD_P - N_P), (0, 3)))
    wp1 = jnp.pad(p['pg1_W'], ((0, 3), (0, 0)))

    sc_deg_m = _make_sc_deg(NPAD_M, NBUCK_M, CH_M, CAP_M)
    sc_deg_p = _make_sc_deg(NPAD_P, NBUCK_P, CH_P, CAP_P)
    sc_gcn_m = _make_sc_gcn(NPAD_M, NBUCK_M, CH_M, CAP_M)
    sc_gcn_p = _make_sc_gcn(NPAD_P, NBUCK_P, CH_P, CAP_P)
    sc_gat_m = _make_sc_gat(NPAD_M, NBUCK_M, CH_M, CAP_M)
    sc_gat_p = _make_sc_gat(NPAD_P, NBUCK_P, CH_P, CAP_P)

    deg_m = sc_deg_m(edstl_m, meta_m)
    deg_p = sc_deg_p(edstl_p, meta_p)
    dinv_m = _tc_dinv(deg_m.reshape(NPAD_M // HID, HID), N_M).reshape(NPAD_M, 1)
    dinv_p = _tc_dinv(deg_p.reshape(NPAD_P // HID, HID), N_P).reshape(NPAD_P, 1)

    lin1 = _tc_gcn_layer(NPAD_M, 8, True)
    linm = _tc_gcn_layer(NPAD_M, HID, False)
    togat_m = _tc_to_gat(NPAD_M, False)
    gat2gat_m = _tc_to_gat(NPAD_M, True)
    gatfin_m = _tc_gat_final(NPAD_M)

    hs1 = lin1(xm, dinv_m, p['mg1_W'])
    acc1 = sc_gcn_m(hs1, esrc_m, edstl_m, meta_m)
    hs2 = linm(acc1, hs1, dinv_m, p['mg1_b'][None, :], p['mg2_W'])
    acc2 = sc_gcn_m(hs2, esrc_m, edstl_m, meta_m)
    hs3 = linm(acc2, hs2, dinv_m, p['mg2_b'][None, :], p['mg3_W'])
    acc3 = sc_gcn_m(hs3, esrc_m, edstl_m, meta_m)
    A1 = _attn_mat(p['ga1_as'], p['ga1_ad'])
    hx4, esed4 = togat_m(acc3, hs3, dinv_m, p['mg3_b'][None, :], p['ga1_W'], A1)
    acc4, aex4 = sc_gat_m(hx4, esed4, esrc_m, edstl_m, meta_m)
    A2 = _attn_mat(p['ga2_as'], p['ga2_ad'])
    hx5, esed5 = gat2gat_m(acc4, aex4, hx4, p['ga1_b'][None, :], p['ga2_W'], A2)
    acc5, aex5 = sc_gat_m(hx5, esed5, esrc_m, edstl_m, meta_m)
    hfin = gatfin_m(acc5, aex5, hx5, p['ga2_b'][None, :])
    msum, mmax, cnt = _tc_pool(hfin, starts)

    linp1 = _tc_gcn_layer(NPAD_P, 8, True)
    linpm = _tc_gcn_layer(NPAD_P, HID, False)
    togat_p = _tc_to_gat(NPAD_P, False)
    gatfin_p = _tc_gat_final(NPAD_P)

    hp1 = linp1(xp, dinv_p, wp1)
    accp1 = sc_gcn_p(hp1, esrc_p, edstl_p, meta_p)
    hp2 = linpm(accp1, hp1, dinv_p, p['pg1_b'][None, :], p['pg2_W'])
    accp2 = sc_gcn_p(hp2, esrc_p, edstl_p, meta_p)
    Ap = _attn_mat(p['pga_as'], p['pga_ad'])
    hxp, esedp = togat_p(accp2, hp2, dinv_p, p['pg2_b'][None, :], p['pga_W'], Ap)
    accp, aexp = sc_gat_p(hxp, esedp, esrc_p, edstl_p, meta_p)
    pfin = gatfin_p(accp, aexp, hxp, p['pga_b'][None, :])
    psum = _tc_psum(pfin, N_P)

    return _tc_head(msum, mmax, cnt, psum, p)
